# baseline (speedup 1.0000x reference)
import functools

import jax
import jax.numpy as jnp
from jax import lax
from jax.experimental import pallas as pl
from jax.experimental.pallas import tpu as pltpu
from jax.experimental.pallas import tpu_sc as plsc

_N = 8192
_D = 256
_E = 8192
_DEPTH = 4
_T = 512
_C = 2048
_CHUNK = 128
_BIG = 2**30


def _dist_argmin_body(r_ref, cb_ref, idx_ref, bva_ref, bia_ref, bvb_ref, bib_ref,
                      *, exact_merge):
    j = pl.program_id(1)
    r = r_ref[...]
    cb = cb_ref[...]
    ones = jnp.ones((8, _D), jnp.float32)
    csq8 = lax.dot_general(ones, cb * cb, (((1,), (1,)), ((), ())),
                           preferred_element_type=jnp.float32,
                           precision=lax.Precision.HIGHEST)
    rsq = jnp.sum(r * r, axis=1, keepdims=True)
    prod = lax.dot_general(r.astype(jnp.bfloat16), cb.astype(jnp.bfloat16),
                           (((1,), (1,)), ((), ())),
                           preferred_element_type=jnp.float32)
    scores = (rsq + csq8[0:1, :]) - 2.0 * prod
    lmin = jnp.min(scores, axis=1, keepdims=True)
    cols = lax.broadcasted_iota(jnp.int32, scores.shape, 1) + j * _C
    larg = jnp.min(jnp.where(scores == lmin, cols, _BIG),
                   axis=1, keepdims=True)

    @pl.when(j == 0)
    def _():
        bva_ref[...] = lmin
        bia_ref[...] = larg

    @pl.when(j == 1)
    def _():
        better = lmin < bva_ref[...]
        bia_ref[...] = jnp.where(better, larg, bia_ref[...])
        bva_ref[...] = jnp.where(better, lmin, bva_ref[...])

    @pl.when(j == 2)
    def _():
        bvb_ref[...] = lmin
        bib_ref[...] = larg

    @pl.when(j == 3)
    def _():
        better = lmin < bvb_ref[...]
        bib = jnp.where(better, larg, bib_ref[...])
        bvb = jnp.where(better, lmin, bvb_ref[...])
        thr = bva_ref[...]
        if not exact_merge:
            thr = thr.astype(jnp.bfloat16).astype(jnp.float32)
        idx_ref[...] = jnp.where(bvb < thr, bib, bia_ref[...])


def _nearest_idx(resid, cb, exact_merge):
    return pl.pallas_call(
        functools.partial(_dist_argmin_body, exact_merge=exact_merge),
        grid=(_N // _T, _E // _C),
        in_specs=[
            pl.BlockSpec((_T, _D), lambda i, j: (i, 0)),
            pl.BlockSpec((_C, _D), lambda i, j: (j, 0)),
        ],
        out_specs=pl.BlockSpec((_T, 1), lambda i, j: (i, 0)),
        out_shape=jax.ShapeDtypeStruct((_N, 1), jnp.int32),
        scratch_shapes=[
            pltpu.VMEM((_T, 1), jnp.float32),
            pltpu.VMEM((_T, 1), jnp.int32),
            pltpu.VMEM((_T, 1), jnp.float32),
            pltpu.VMEM((_T, 1), jnp.int32),
        ],
        compiler_params=pltpu.CompilerParams(
            dimension_semantics=("arbitrary", "arbitrary")),
    )(resid, cb)


@functools.cache
def _make_sc_update(with_quants: bool):
    info = plsc.get_sparse_core_info()
    nc, ns, nl = info.num_cores, info.num_subcores, info.num_lanes
    nw = nc * ns
    per_w = _N // nw
    n_chunks = per_w // _CHUNK
    mesh = plsc.VectorSubcoreMesh(core_axis_name="c", subcore_axis_name="s")

    out_type = [jax.ShapeDtypeStruct((_N, _D), jnp.float32),
                jax.ShapeDtypeStruct((nw, nl), jnp.float32)]
    scratch = [pltpu.VMEM((_CHUNK,), jnp.int32),
               pltpu.VMEM((_CHUNK, _D), jnp.float32),
               pltpu.VMEM((_CHUNK, _D), jnp.float32)]
    if with_quants:
        scratch.append(pltpu.VMEM((_CHUNK, _D), jnp.float32))
    scratch += [pltpu.VMEM((nl,), jnp.float32),
                pltpu.SemaphoreType.DMA]

    @functools.partial(pl.kernel, mesh=mesh, out_type=out_type,
                       scratch_types=scratch)
    def sc_update(*refs):
        if with_quants:
            (cb_hbm, idx_hbm, r_hbm, x_hbm, out_hbm, loss_hbm,
             idx_v, q_v, r_v, x_v, loss_v, sem) = refs
        else:
            (cb_hbm, idx_hbm, r_hbm, out_hbm, loss_hbm,
             idx_v, q_v, r_v, loss_v, sem) = refs
        wid = lax.axis_index("s") * nc + lax.axis_index("c")
        acc = jnp.zeros((nl,), jnp.float32)
        for k in range(n_chunks):
            base = wid * per_w + k * _CHUNK
            pltpu.sync_copy(idx_hbm.at[pl.ds(base, _CHUNK)], idx_v)
            pltpu.async_copy(cb_hbm.at[idx_v], q_v, sem).wait()
            pltpu.sync_copy(r_hbm.at[pl.ds(base, _CHUNK), :], r_v)
            if with_quants:
                pltpu.sync_copy(x_hbm.at[pl.ds(base, _CHUNK), :], x_v)

            def tok_body(t, a):
                for c in range(_D // nl):
                    sl = pl.ds(c * nl, nl)
                    newv = r_v[t, sl] - q_v[t, sl]
                    a = a + newv * newv
                    if with_quants:
                        r_v[t, sl] = x_v[t, sl] - newv
                    else:
                        r_v[t, sl] = newv
                return a

            acc = lax.fori_loop(0, _CHUNK, tok_body, acc)
            pltpu.sync_copy(r_v, out_hbm.at[pl.ds(base, _CHUNK), :])
        loss_v[...] = acc
        pltpu.sync_copy(loss_v, loss_hbm.at[wid])

    return sc_update


def kernel(x, codebooks):
    b, h, w, d = x.shape
    x_flat = x.reshape(-1, d)
    resid = x_flat
    idxs = []
    loss_parts = []
    sc_mid = _make_sc_update(False)
    sc_last = _make_sc_update(True)
    quants_flat = None
    for lvl in range(_DEPTH):
        cb = codebooks[lvl]
        idx2d = _nearest_idx(resid, cb, exact_merge=(lvl == 0))
        idx = idx2d.reshape(-1)
        idxs.append(idx2d)
        if lvl < _DEPTH - 1:
            resid, lp = sc_mid(cb, idx, resid)
        else:
            quants_flat, lp = sc_last(cb, idx, resid, x_flat)
        loss_parts.append(jnp.sum(lp) / jnp.float32(_N * _D))
    commitment_loss = jnp.mean(jnp.stack(loss_parts))
    quants = quants_flat.reshape(x.shape)
    codes = jnp.concatenate(idxs, axis=1).reshape(b, h, w, _DEPTH)
    return quants, commitment_loss, codes


# speedup vs baseline: 1.9614x; 1.9614x over previous
import functools

import jax
import jax.numpy as jnp
from jax import lax
from jax.experimental import pallas as pl
from jax.experimental.pallas import tpu as pltpu
from jax.experimental.pallas import tpu_sc as plsc

_N = 8192
_D = 256
_E = 8192
_DEPTH = 4
_T = 512
_C = 2048
_H = 4096
_CHUNK = 128
_BIG = 2**30


def _csq_body(cb_ref, csq_ref):
    ones = jnp.ones((8, _D), jnp.float32)
    csq_ref[...] = lax.dot_general(ones, cb_ref[...] * cb_ref[...],
                                   (((1,), (1,)), ((), ())),
                                   preferred_element_type=jnp.float32,
                                   precision=lax.Precision.HIGHEST)


def _csq_row(cb):
    return pl.pallas_call(
        _csq_body,
        grid=(_E // _C,),
        in_specs=[pl.BlockSpec((_C, _D), lambda j: (j, 0))],
        out_specs=pl.BlockSpec((8, _C), lambda j: (0, j)),
        out_shape=jax.ShapeDtypeStruct((8, _E), jnp.float32),
        compiler_params=pltpu.CompilerParams(
            dimension_semantics=("arbitrary",)),
    )(cb)


def _dist_argmin_body(r_ref, cb_ref, csq_ref, idx_ref,
                      rsq_ref, bva_ref, bia_ref, *, exact_merge):
    j = pl.program_id(1)
    r = r_ref[...]

    @pl.when(j == 0)
    def _():
        rsq_ref[...] = jnp.sum(r * r, axis=1, keepdims=True)

    prod = lax.dot_general(r.astype(jnp.bfloat16),
                           cb_ref[...].astype(jnp.bfloat16),
                           (((1,), (1,)), ((), ())),
                           preferred_element_type=jnp.float32)
    scores = (rsq_ref[...] + csq_ref[0:1, :]) - 2.0 * prod
    lmin = jnp.min(scores, axis=1, keepdims=True)
    cols = lax.broadcasted_iota(jnp.int32, scores.shape, 1) + j * _H
    larg = jnp.min(jnp.where(scores == lmin, cols, _BIG),
                   axis=1, keepdims=True)

    @pl.when(j == 0)
    def _():
        bva_ref[...] = lmin
        bia_ref[...] = larg

    @pl.when(j == 1)
    def _():
        thr = bva_ref[...]
        if not exact_merge:
            thr = thr.astype(jnp.bfloat16).astype(jnp.float32)
        idx_ref[...] = jnp.where(lmin < thr, larg, bia_ref[...])


def _nearest_idx(resid, cb, exact_merge):
    csq = _csq_row(cb)
    return pl.pallas_call(
        functools.partial(_dist_argmin_body, exact_merge=exact_merge),
        grid=(_N // _T, _E // _H),
        in_specs=[
            pl.BlockSpec((_T, _D), lambda i, j: (i, 0)),
            pl.BlockSpec((_H, _D), lambda i, j: (j, 0)),
            pl.BlockSpec((8, _H), lambda i, j: (0, j)),
        ],
        out_specs=pl.BlockSpec((_T, 1), lambda i, j: (i, 0)),
        out_shape=jax.ShapeDtypeStruct((_N, 1), jnp.int32),
        scratch_shapes=[
            pltpu.VMEM((_T, 1), jnp.float32),
            pltpu.VMEM((_T, 1), jnp.float32),
            pltpu.VMEM((_T, 1), jnp.int32),
        ],
        compiler_params=pltpu.CompilerParams(
            dimension_semantics=("arbitrary", "arbitrary")),
    )(resid, cb, csq)


@functools.cache
def _make_sc_update(with_quants: bool):
    info = plsc.get_sparse_core_info()
    nc, ns, nl = info.num_cores, info.num_subcores, info.num_lanes
    nw = nc * ns
    per_w = _N // nw
    n_chunks = per_w // _CHUNK
    mesh = plsc.VectorSubcoreMesh(core_axis_name="c", subcore_axis_name="s")

    out_type = [jax.ShapeDtypeStruct((_N, _D), jnp.float32),
                jax.ShapeDtypeStruct((nw, nl), jnp.float32)]
    scratch = [pltpu.VMEM((_CHUNK,), jnp.int32),
               pltpu.VMEM((_CHUNK, _D), jnp.float32),
               pltpu.VMEM((_CHUNK, _D), jnp.float32)]
    if with_quants:
        scratch.append(pltpu.VMEM((_CHUNK, _D), jnp.float32))
    scratch += [pltpu.VMEM((nl,), jnp.float32),
                pltpu.SemaphoreType.DMA]

    @functools.partial(pl.kernel, mesh=mesh, out_type=out_type,
                       scratch_types=scratch)
    def sc_update(*refs):
        if with_quants:
            (cb_hbm, idx_hbm, r_hbm, x_hbm, out_hbm, loss_hbm,
             idx_v, q_v, r_v, x_v, loss_v, sem) = refs
        else:
            (cb_hbm, idx_hbm, r_hbm, out_hbm, loss_hbm,
             idx_v, q_v, r_v, loss_v, sem) = refs
        wid = lax.axis_index("s") * nc + lax.axis_index("c")
        acc = jnp.zeros((nl,), jnp.float32)
        for k in range(n_chunks):
            base = wid * per_w + k * _CHUNK
            pltpu.sync_copy(idx_hbm.at[pl.ds(base, _CHUNK)], idx_v)
            pltpu.async_copy(cb_hbm.at[idx_v], q_v, sem).wait()
            pltpu.sync_copy(r_hbm.at[pl.ds(base, _CHUNK), :], r_v)
            if with_quants:
                pltpu.sync_copy(x_hbm.at[pl.ds(base, _CHUNK), :], x_v)

            def tok_body(t, a):
                for c in range(_D // nl):
                    sl = pl.ds(c * nl, nl)
                    newv = r_v[t, sl] - q_v[t, sl]
                    a = a + newv * newv
                    if with_quants:
                        r_v[t, sl] = x_v[t, sl] - newv
                    else:
                        r_v[t, sl] = newv
                return a

            acc = lax.fori_loop(0, _CHUNK, tok_body, acc)
            pltpu.sync_copy(r_v, out_hbm.at[pl.ds(base, _CHUNK), :])
        loss_v[...] = acc
        pltpu.sync_copy(loss_v, loss_hbm.at[wid])

    return sc_update


def kernel(x, codebooks):
    b, h, w, d = x.shape
    x_flat = x.reshape(-1, d)
    resid = x_flat
    idxs = []
    loss_parts = []
    sc_mid = _make_sc_update(False)
    sc_last = _make_sc_update(True)
    quants_flat = None
    for lvl in range(_DEPTH):
        cb = codebooks[lvl]
        idx2d = _nearest_idx(resid, cb, exact_merge=(lvl == 0))
        idx = idx2d.reshape(-1)
        idxs.append(idx2d)
        if lvl < _DEPTH - 1:
            resid, lp = sc_mid(cb, idx, resid)
        else:
            quants_flat, lp = sc_last(cb, idx, resid, x_flat)
        loss_parts.append(jnp.sum(lp) / jnp.float32(_N * _D))
    commitment_loss = jnp.mean(jnp.stack(loss_parts))
    quants = quants_flat.reshape(x.shape)
    codes = jnp.concatenate(idxs, axis=1).reshape(b, h, w, _DEPTH)
    return quants, commitment_loss, codes
